# baseline (speedup 1.0000x reference)
import jax
import jax.numpy as jnp
from jax import lax
from jax.experimental import pallas as pl
from jax.experimental.pallas import tpu as pltpu
from jax.experimental.pallas import tpu_sc as plsc

N_EV, N_CS, N_TR, N_ENV = 60000, 20000, 15000, 5000
NN = N_EV + N_CS + N_TR + N_ENV
NE = 1600000
FD, HD = 8, 32
MAX_ACTION = 1.0

NC, NS = 2, 16
NWORK = NC * NS
EPW = NE // NWORK
CHUNK = 80
NCHUNK = EPW // CHUNK
G = 25
NP = 100096
STRIPE = NP // NS

NR = NP // 16
BR = 368
GRID = NR // BR

_MESH = plsc.VectorSubcoreMesh(
    core_axis_name="c", subcore_axis_name="s", num_cores=NC, num_subcores=NS)


def _agg_body(src2_hbm, dst2_hbm, u_hbm, zeros_hbm, out_hbm,
              sidx, didx, rows, acc, isem, gsem, ssem):
  c = lax.axis_index("c")
  s = lax.axis_index("s")
  wid = s * NC + c
  rowbase0 = wid * NCHUNK
  row0 = s * STRIPE
  pltpu.sync_copy(zeros_hbm.at[pl.ds(row0, STRIPE)],
                  acc.at[pl.ds(row0, STRIPE)])
  plsc.subcore_barrier()

  def step(k, carry):
    rb = rowbase0 + k * G
    ds_ = pltpu.async_copy(src2_hbm.at[pl.ds(rb, G)], sidx, isem)
    dd_ = pltpu.async_copy(dst2_hbm.at[pl.ds(rb, G)], didx, isem)
    ds_.wait()
    dd_.wait()
    gd = [pltpu.async_copy(u_hbm.at[sidx.at[g]], rows.at[g], gsem)
          for g in range(G)]
    for d in gd:
      d.wait()
    sd = [pltpu.async_copy(rows.at[g], acc.at[didx.at[g]], ssem, add=True)
          for g in range(G)]
    for d in sd:
      d.wait()
    return carry

  lax.fori_loop(0, NCHUNK // G, step, 0)
  plsc.subcore_barrier()
  pltpu.sync_copy(acc.at[pl.ds(row0, STRIPE)],
                  out_hbm.at[c, pl.ds(row0, STRIPE)])


_agg_call = pl.kernel(
    _agg_body,
    out_type=jax.ShapeDtypeStruct((NC, NP, FD), jnp.float32),
    mesh=_MESH,
    scratch_types=[
        pltpu.VMEM((G, CHUNK), jnp.int32),
        pltpu.VMEM((G, CHUNK), jnp.int32),
        pltpu.VMEM((G, CHUNK, FD), jnp.float32),
        pltpu.VMEM_SHARED((NP, FD), jnp.float32),
        pltpu.SemaphoreType.DMA,
        pltpu.SemaphoreType.DMA,
        pltpu.SemaphoreType.DMA,
    ],
    compiler_params=pltpu.CompilerParams(use_tc_tiling_on_sc=False),
    name="sc_edge_agg",
)


def _deg_body(dst2_hbm, ones_hbm, zeros_hbm, out_hbm, didx, ones_v, acc,
              isem, ssem):
  c = lax.axis_index("c")
  s = lax.axis_index("s")
  wid = s * NC + c
  rowbase0 = wid * NCHUNK
  row0 = s * STRIPE
  pltpu.sync_copy(ones_hbm, ones_v)
  pltpu.sync_copy(zeros_hbm.at[pl.ds(row0, STRIPE)],
                  acc.at[pl.ds(row0, STRIPE)])
  plsc.subcore_barrier()

  def step(k, carry):
    rb = rowbase0 + k * G
    pltpu.async_copy(dst2_hbm.at[pl.ds(rb, G)], didx, isem).wait()
    sd = [pltpu.async_copy(ones_v, acc.at[didx.at[g]], ssem, add=True)
          for g in range(G)]
    for d in sd:
      d.wait()
    return carry

  lax.fori_loop(0, NCHUNK // G, step, 0)
  plsc.subcore_barrier()
  pltpu.sync_copy(acc.at[pl.ds(row0, STRIPE)],
                  out_hbm.at[c, pl.ds(row0, STRIPE)])


_deg_call = pl.kernel(
    _deg_body,
    out_type=jax.ShapeDtypeStruct((NC, NP, FD), jnp.float32),
    mesh=_MESH,
    scratch_types=[
        pltpu.VMEM((G, CHUNK), jnp.int32),
        pltpu.VMEM((CHUNK, FD), jnp.float32),
        pltpu.VMEM_SHARED((NP, FD), jnp.float32),
        pltpu.SemaphoreType.DMA,
        pltpu.SemaphoreType.DMA,
    ],
    compiler_params=pltpu.CompilerParams(use_tc_tiling_on_sc=False),
    name="sc_degree",
)


def _tc_embed_body(featpp_ref, me_ref, bp_ref, d_ref, b8_ref,
                   dinv_ref, u1_ref):
  d = d_ref[...]
  degb = jnp.dot(d[0] + d[1], b8_ref[...],
                 preferred_element_type=jnp.float32)
  dinv = lax.rsqrt(degb + 1.0)
  emb = jnp.dot(featpp_ref[...], me_ref[...],
                preferred_element_type=jnp.float32) + bp_ref[...]
  x0 = jnp.maximum(emb, 0.0)
  dinv_ref[...] = dinv
  u1_ref[...] = x0 * dinv


def _tc_dense1_body(p_ref, u1_ref, dinv_ref, m1_ref, b1_ref, m2_ref,
                    u2_ref):
  p = p_ref[...]
  dinv = dinv_ref[...]
  agg = (p[0] + p[1] + u1_ref[...]) * dinv
  x1 = jnp.maximum(
      jnp.dot(agg, m1_ref[...], preferred_element_type=jnp.float32)
      + b1_ref[...], 0.0)
  u2_ref[...] = jnp.dot(
      x1, m2_ref[...], preferred_element_type=jnp.float32) * dinv


def _tc_dense2_body(q_ref, u2_ref, dinv_ref, b2_ref, m3_ref, u3_ref):
  q = q_ref[...]
  dinv = dinv_ref[...]
  x2 = jnp.maximum((q[0] + q[1] + u2_ref[...]) * dinv + b2_ref[...], 0.0)
  u3_ref[...] = jnp.dot(
      x2, m3_ref[...], preferred_element_type=jnp.float32) * dinv


def _tc_final_body(r_ref, u3_ref, dinv_ref, s_ref, bl_ref, out_ref):
  r = r_ref[...]
  v = (r[0] + r[1] + u3_ref[...]) * dinv_ref[...]
  sel = jnp.dot(v, s_ref[...], preferred_element_type=jnp.float32)
  out_ref[...] = MAX_ACTION * jnp.tanh(sel + bl_ref[...])


def _full(shape):
  return pl.BlockSpec(shape, lambda i: (0,) * len(shape))


_prow = pl.BlockSpec((BR, 128), lambda i: (i, 0))
_ppar = pl.BlockSpec((NC, BR, 128), lambda i: (0, i, 0))

_tc_embed = pl.pallas_call(
    _tc_embed_body,
    grid=(GRID,),
    in_specs=[pl.BlockSpec((BR, 1024), lambda i: (i, 0)),
              _full((1024, 128)), _prow, _ppar, _full((128, 128))],
    out_specs=[_prow, _prow],
    out_shape=[
        jax.ShapeDtypeStruct((NR, 128), jnp.float32),
        jax.ShapeDtypeStruct((NR, 128), jnp.float32),
    ],
)

_tc_dense1 = pl.pallas_call(
    _tc_dense1_body,
    grid=(GRID,),
    in_specs=[_ppar, _prow, _prow, _full((128, 512)), _full((1, 512)),
              _full((512, 128))],
    out_specs=_prow,
    out_shape=jax.ShapeDtypeStruct((NR, 128), jnp.float32),
)

_tc_dense2 = pl.pallas_call(
    _tc_dense2_body,
    grid=(GRID,),
    in_specs=[_ppar, _prow, _prow, _full((1, 128)), _full((128, 128))],
    out_specs=_prow,
    out_shape=jax.ShapeDtypeStruct((NR, 128), jnp.float32),
)

_tc_final = pl.pallas_call(
    _tc_final_body,
    grid=(GRID,),
    in_specs=[_ppar, _prow, _prow, _full((128, 16)), _full((1, 1))],
    out_specs=pl.BlockSpec((BR, 16), lambda i: (i, 0)),
    out_shape=jax.ShapeDtypeStruct((NR, 16), jnp.float32),
)


def kernel(ev_features, cs_features, tr_features, env_features, edge_index,
           ev_indexes, cs_indexes, tr_indexes, env_indexes,
           W_ev, b_ev, W_cs, b_cs, W_tr, b_tr, W_env, b_env,
           W_g1, b_g1, W_g2, b_g2, W_gl, b_gl):
  f32 = jnp.float32
  src2 = edge_index[0].reshape(NE // CHUNK, CHUNK)
  dst2 = edge_index[1].reshape(NE // CHUNK, CHUNK)

  feat16 = jnp.concatenate([
      ev_features,
      jnp.pad(cs_features, ((0, 0), (0, 4))),
      jnp.pad(tr_features, ((0, 0), (0, 6))),
      jnp.pad(env_features, ((0, 0), (0, 8))),
      jnp.zeros((NP - NN, 16), f32),
  ], axis=0)
  tv = jnp.concatenate([
      jnp.full((N_EV,), 0, jnp.int32), jnp.full((N_CS,), 1, jnp.int32),
      jnp.full((N_TR,), 2, jnp.int32),
      jnp.full((N_ENV + NP - NN,), 3, jnp.int32)])
  oh = jax.nn.one_hot(tv, 4, dtype=f32)
  featpp = (oh[:, :, None] * feat16[:, None, :]).reshape(NR, 1024)

  eye16 = jnp.eye(16, dtype=f32)
  wall = jnp.concatenate([
      W_ev,
      jnp.pad(W_cs, ((0, 4), (0, 0))),
      jnp.pad(W_tr, ((0, 6), (0, 0))),
      jnp.pad(W_env, ((0, 8), (0, 0))),
  ], axis=0)
  me = jnp.kron(eye16, wall)
  bstack = jnp.stack([b_ev, b_cs, b_tr, b_env])
  bp = bstack[tv].reshape(NR, 128)
  b8 = jnp.kron(eye16, jnp.zeros((8, 8), f32).at[0].set(1.0))
  m1 = jnp.kron(eye16, W_g1)
  m2 = jnp.kron(eye16, W_g2)
  e0row = jnp.zeros((1, FD), f32).at[0, 0].set(1.0)
  m3 = jnp.kron(eye16, W_gl @ e0row)
  sel = jnp.kron(eye16, jnp.zeros((FD, 1), f32).at[0].set(1.0))
  b1row = jnp.tile(b_g1, 16)[None, :]
  b2row = jnp.tile(b_g2, 16)[None, :]

  zeros = jnp.zeros((NP, FD), f32)
  ones_rows = jnp.zeros((CHUNK, FD), f32).at[:, 0].set(1.0)

  degp = _deg_call(dst2, ones_rows, zeros).reshape(NC, NR, 128)
  dinvp, u1p = _tc_embed(featpp, me, bp, degp, b8)
  p = _agg_call(src2, dst2, u1p.reshape(NP, FD), zeros).reshape(NC, NR, 128)
  u2p = _tc_dense1(p, u1p, dinvp, m1, b1row, m2)
  q = _agg_call(src2, dst2, u2p.reshape(NP, FD), zeros).reshape(NC, NR, 128)
  u3p = _tc_dense2(q, u2p, dinvp, b2row, m3)
  r = _agg_call(src2, dst2, u3p.reshape(NP, FD), zeros).reshape(NC, NR, 128)
  outp = _tc_final(r, u3p, dinvp, sel, b_gl.reshape(1, 1))
  return outp.reshape(NP)[:NN]
